# baseline (speedup 1.0000x reference)
import jax
import jax.numpy as jnp
from jax.experimental import pallas as pl
from jax.experimental.pallas import tpu as pltpu

_BM = 400
_HALF = _BM // 2


def _sage_kernel(a0_ref, a1_ref, h_ref, w_ref, out_ref):
    wt = w_ref[...].T
    x0 = jnp.dot(a0_ref[...], h_ref[...], preferred_element_type=jnp.float32)
    out_ref[:_HALF, :] = jnp.dot(x0, wt, preferred_element_type=jnp.float32)
    x1 = jnp.dot(a1_ref[...], h_ref[...], preferred_element_type=jnp.float32)
    out_ref[_HALF:, :] = jnp.dot(x1, wt, preferred_element_type=jnp.float32)


def kernel(adj, h, W):
    n, _ = adj.shape
    d_in = h.shape[1]
    d_out = W.shape[0]
    grid = (pl.cdiv(n, _BM),)
    return pl.pallas_call(
        _sage_kernel,
        grid=grid,
        in_specs=[
            pl.BlockSpec((_HALF, n), lambda i: (2 * i, 0)),
            pl.BlockSpec((_HALF, n), lambda i: (2 * i + 1, 0)),
            pl.BlockSpec((n, d_in), lambda i: (0, 0)),
            pl.BlockSpec((d_out, d_in), lambda i: (0, 0)),
        ],
        out_specs=pl.BlockSpec((_BM, d_out), lambda i: (i, 0)),
        out_shape=jax.ShapeDtypeStruct((n, d_out), jnp.float32),
        compiler_params=pltpu.CompilerParams(
            dimension_semantics=("parallel",)),
    )(adj, adj, h, W)


# speedup vs baseline: 1.0660x; 1.0660x over previous
import jax
import jax.numpy as jnp
from jax.experimental import pallas as pl
from jax.experimental.pallas import tpu as pltpu

_BM = 600


def _sage_kernel(adj_ref, h_ref, w_ref, out_ref):
    x = jnp.dot(adj_ref[...], h_ref[...], preferred_element_type=jnp.float32)
    out_ref[...] = jax.lax.dot_general(
        x, w_ref[...], (((1,), (1,)), ((), ())),
        preferred_element_type=jnp.float32)


def kernel(adj, h, W):
    n, _ = adj.shape
    d_in = h.shape[1]
    d_out = W.shape[0]
    grid = (pl.cdiv(n, _BM),)
    return pl.pallas_call(
        _sage_kernel,
        grid=grid,
        in_specs=[
            pl.BlockSpec((_BM, n), lambda i: (i, 0)),
            pl.BlockSpec((n, d_in), lambda i: (0, 0)),
            pl.BlockSpec((d_out, d_in), lambda i: (0, 0)),
        ],
        out_specs=pl.BlockSpec((_BM, d_out), lambda i: (i, 0)),
        out_shape=jax.ShapeDtypeStruct((n, d_out), jnp.float32),
        compiler_params=pltpu.CompilerParams(
            dimension_semantics=("parallel",)),
    )(adj, h, W)
